# baseline (device time: 123345 ns/iter reference)
import jax
import jax.numpy as jnp
from jax import lax
from jax.experimental import pallas as pl
from jax.experimental.pallas import tpu as pltpu

N_DEV = 16
M = 2048
N = 2048
M_PER = M // N_DEV


def _gelu(y):
    c = 0.7978845608028654
    return 0.5 * y * (1.0 + jnp.tanh(c * (y + 0.044715 * y * y * y)))


def kernel(x, w_mat):
    def body(x_ref, w_ref, out_ref, part_ref, send_ref, recv_ref,
             send_sems, recv_sems):
        d = lax.axis_index("i")
        right = lax.rem(d + 1, N_DEV)
        left = lax.rem(d + N_DEV - 1, N_DEV)

        barrier_sem = pltpu.get_barrier_semaphore()
        for nbr in (left, right):
            pl.semaphore_signal(
                barrier_sem, inc=1,
                device_id=(nbr,), device_id_type=pl.DeviceIdType.MESH,
            )
        pl.semaphore_wait(barrier_sem, 2)

        part_ref[...] = jnp.dot(
            x_ref[...], w_ref[...], preferred_element_type=jnp.float32
        ).astype(jnp.bfloat16)

        def chunk(c):
            return part_ref[pl.ds(c * M_PER, M_PER), :]

        send_ref[0] = chunk(lax.rem(d + N_DEV - 1, N_DEV))

        for s in range(N_DEV - 1):
            rdma = pltpu.make_async_remote_copy(
                src_ref=send_ref.at[s],
                dst_ref=recv_ref.at[s],
                send_sem=send_sems.at[s],
                recv_sem=recv_sems.at[s],
                device_id=(right,),
                device_id_type=pl.DeviceIdType.MESH,
            )
            rdma.start()
            rdma.wait()
            if s < N_DEV - 2:
                c = lax.rem(d + N_DEV - 2 - s, N_DEV)
                send_ref[s + 1] = (
                    recv_ref[s].astype(jnp.float32)
                    + chunk(c).astype(jnp.float32)
                ).astype(jnp.bfloat16)

        y = (recv_ref[N_DEV - 2].astype(jnp.float32)
             + chunk(d).astype(jnp.float32))
        out_ref[...] = _gelu(y)

    return pl.pallas_call(
        body,
        out_shape=jax.ShapeDtypeStruct((M_PER, N), jnp.float32),
        in_specs=[
            pl.BlockSpec(memory_space=pltpu.VMEM),
            pl.BlockSpec(memory_space=pltpu.VMEM),
        ],
        out_specs=pl.BlockSpec(memory_space=pltpu.VMEM),
        scratch_shapes=[
            pltpu.VMEM((M, N), jnp.bfloat16),
            pltpu.VMEM((N_DEV - 1, M_PER, N), jnp.bfloat16),
            pltpu.VMEM((N_DEV - 1, M_PER, N), jnp.bfloat16),
            pltpu.SemaphoreType.DMA((N_DEV - 1,)),
            pltpu.SemaphoreType.DMA((N_DEV - 1,)),
        ],
        compiler_params=pltpu.CompilerParams(collective_id=0),
    )(x, w_mat)


# device time: 67427 ns/iter; 1.8293x vs baseline; 1.8293x over previous
import jax
import jax.numpy as jnp
from jax import lax
from jax.experimental import pallas as pl
from jax.experimental.pallas import tpu as pltpu

N_DEV = 16
M = 2048
N = 2048
M_PER = M // N_DEV
HALF = N // 2
G = 4
GW = HALF // G
N_HOP = N_DEV - 1


def _gelu(y):
    c = 0.7978845608028654
    return 0.5 * y * (1.0 + jnp.tanh(c * (y + 0.044715 * y * y * y)))


def kernel(x, w_mat):
    def body(x_ref, w_ref, out_ref, part_ref, send_ref, recv_ref,
             send_sems, recv_sems):
        d = lax.axis_index("i")
        right = lax.rem(d + 1, N_DEV)
        left = lax.rem(d + N_DEV - 1, N_DEV)

        barrier_sem = pltpu.get_barrier_semaphore()
        for nbr in (left, right):
            pl.semaphore_signal(
                barrier_sem, inc=1,
                device_id=(nbr,), device_id_type=pl.DeviceIdType.MESH,
            )
        pl.semaphore_wait(barrier_sem, 2)

        part_ref[...] = jnp.dot(
            x_ref[...], w_ref[...], preferred_element_type=jnp.float32
        ).astype(jnp.bfloat16)

        def chunk_cols(c, dirn, g):
            col0 = dirn * HALF + g * GW
            return part_ref[pl.ds(c * M_PER, M_PER), col0:col0 + GW]

        def send_chunk_idx(dirn, s):
            if dirn == 0:
                return lax.rem(d + 2 * N_DEV - 1 - s, N_DEV)
            return lax.rem(d + 1 + s, N_DEV)

        def rdma(s, dirn, g):
            return pltpu.make_async_remote_copy(
                src_ref=send_ref.at[s, dirn, g],
                dst_ref=recv_ref.at[s, dirn, g],
                send_sem=send_sems.at[s, dirn, g],
                recv_sem=recv_sems.at[s, dirn, g],
                device_id=(right if dirn == 0 else left,),
                device_id_type=pl.DeviceIdType.MESH,
            )

        for dirn in (0, 1):
            for g in range(G):
                send_ref[0, dirn, g] = chunk_cols(send_chunk_idx(dirn, 0),
                                                  dirn, g)

        for s in range(N_HOP):
            for g in range(G):
                for dirn in (0, 1):
                    if s > 0:
                        rdma(s - 1, dirn, g).wait_recv()
                        send_ref[s, dirn, g] = (
                            recv_ref[s - 1, dirn, g].astype(jnp.float32)
                            + chunk_cols(send_chunk_idx(dirn, s), dirn, g)
                            .astype(jnp.float32)
                        ).astype(jnp.bfloat16)
                    rdma(s, dirn, g).start()

        for dirn in (0, 1):
            for g in range(G):
                rdma(N_HOP - 1, dirn, g).wait_recv()
                col0 = dirn * HALF + g * GW
                y = (recv_ref[N_HOP - 1, dirn, g].astype(jnp.float32)
                     + chunk_cols(d, dirn, g).astype(jnp.float32))
                out_ref[:, col0:col0 + GW] = _gelu(y)

        for s in range(N_HOP):
            for dirn in (0, 1):
                for g in range(G):
                    rdma(s, dirn, g).wait_send()

    return pl.pallas_call(
        body,
        out_shape=jax.ShapeDtypeStruct((M_PER, N), jnp.float32),
        in_specs=[
            pl.BlockSpec(memory_space=pltpu.VMEM),
            pl.BlockSpec(memory_space=pltpu.VMEM),
        ],
        out_specs=pl.BlockSpec(memory_space=pltpu.VMEM),
        scratch_shapes=[
            pltpu.VMEM((M, N), jnp.bfloat16),
            pltpu.VMEM((N_HOP, 2, G, M_PER, GW), jnp.bfloat16),
            pltpu.VMEM((N_HOP, 2, G, M_PER, GW), jnp.bfloat16),
            pltpu.SemaphoreType.DMA((N_HOP, 2, G)),
            pltpu.SemaphoreType.DMA((N_HOP, 2, G)),
        ],
        compiler_params=pltpu.CompilerParams(collective_id=0),
    )(x, w_mat)


# device time: 65174 ns/iter; 1.8925x vs baseline; 1.0346x over previous
import jax
import jax.numpy as jnp
from jax import lax
from jax.experimental import pallas as pl
from jax.experimental.pallas import tpu as pltpu

N_DEV = 16
M = 2048
N = 2048
M_PER = M // N_DEV
HALF = N // 2
G = 4
GW = HALF // G
N_HOP = N_DEV - 1


def _gelu(y):
    c = 0.7978845608028654
    return 0.5 * y * (1.0 + jnp.tanh(c * (y + 0.044715 * y * y * y)))


def kernel(x, w_mat):
    def body(x_ref, w_ref, out_ref, part_ref, send_ref, recv_ref,
             send_sems, recv_sems):
        d = lax.axis_index("i")
        right = lax.rem(d + 1, N_DEV)
        left = lax.rem(d + N_DEV - 1, N_DEV)

        barrier_sem = pltpu.get_barrier_semaphore()
        for nbr in (left, right):
            pl.semaphore_signal(
                barrier_sem, inc=1,
                device_id=(nbr,), device_id_type=pl.DeviceIdType.MESH,
            )
        pl.semaphore_wait(barrier_sem, 2)

        def chunk_cols(c, dirn, g):
            col0 = dirn * HALF + g * GW
            return part_ref[pl.ds(c * M_PER, M_PER), col0:col0 + GW]

        def send_chunk_idx(dirn, s):
            if dirn == 0:
                return lax.rem(d + 2 * N_DEV - 1 - s, N_DEV)
            return lax.rem(d + 1 + s, N_DEV)

        def rdma(s, dirn, g):
            return pltpu.make_async_remote_copy(
                src_ref=send_ref.at[s, dirn, g],
                dst_ref=recv_ref.at[s, dirn, g],
                send_sem=send_sems.at[s, dirn, g],
                recv_sem=recv_sems.at[s, dirn, g],
                device_id=(right if dirn == 0 else left,),
                device_id_type=pl.DeviceIdType.MESH,
            )

        for dirn in (0, 1):
            c = send_chunk_idx(dirn, 0)
            p0 = jnp.dot(
                x_ref[pl.ds(c * M_PER, M_PER), :], w_ref[...],
                preferred_element_type=jnp.float32,
            ).astype(jnp.bfloat16)
            for g in range(G):
                col0 = dirn * HALF + g * GW
                send_ref[0, dirn, g] = p0[:, col0:col0 + GW]
            for g in range(G):
                rdma(0, dirn, g).start()

        part_ref[...] = jnp.dot(
            x_ref[...], w_ref[...], preferred_element_type=jnp.float32
        ).astype(jnp.bfloat16)

        for s in range(1, N_HOP):
            for g in range(G):
                for dirn in (0, 1):
                    rdma(s - 1, dirn, g).wait_recv()
                    send_ref[s, dirn, g] = (
                        recv_ref[s - 1, dirn, g]
                        + chunk_cols(send_chunk_idx(dirn, s), dirn, g)
                    )
                    rdma(s, dirn, g).start()

        for dirn in (0, 1):
            for g in range(G):
                rdma(N_HOP - 1, dirn, g).wait_recv()
                col0 = dirn * HALF + g * GW
                y = (recv_ref[N_HOP - 1, dirn, g].astype(jnp.float32)
                     + chunk_cols(d, dirn, g).astype(jnp.float32))
                out_ref[:, col0:col0 + GW] = _gelu(y)

        for s in range(N_HOP):
            for dirn in (0, 1):
                for g in range(G):
                    rdma(s, dirn, g).wait_send()

    return pl.pallas_call(
        body,
        out_shape=jax.ShapeDtypeStruct((M_PER, N), jnp.float32),
        in_specs=[
            pl.BlockSpec(memory_space=pltpu.VMEM),
            pl.BlockSpec(memory_space=pltpu.VMEM),
        ],
        out_specs=pl.BlockSpec(memory_space=pltpu.VMEM),
        scratch_shapes=[
            pltpu.VMEM((M, N), jnp.bfloat16),
            pltpu.VMEM((N_HOP, 2, G, M_PER, GW), jnp.bfloat16),
            pltpu.VMEM((N_HOP, 2, G, M_PER, GW), jnp.bfloat16),
            pltpu.SemaphoreType.DMA((N_HOP, 2, G)),
            pltpu.SemaphoreType.DMA((N_HOP, 2, G)),
        ],
        compiler_params=pltpu.CompilerParams(collective_id=0),
    )(x, w_mat)


# device time: 53586 ns/iter; 2.3018x vs baseline; 1.2163x over previous
import jax
import jax.numpy as jnp
from jax import lax
from jax.experimental import pallas as pl
from jax.experimental.pallas import tpu as pltpu

N_DEV = 16
M = 2048
N = 2048
M_PER = M // N_DEV
HALF = N // 2
G = 4
GW = HALF // G
N_HOP = N_DEV - 1


def _gelu(y):
    c = 0.7978845608028654
    return 0.5 * y * (1.0 + jnp.tanh(c * (y + 0.044715 * y * y * y)))


def kernel(x, w_mat):
    def body(x_ref, w_ref, out_ref, part_ref, send_ref, recv_ref,
             send_sems, recv_sems):
        d = lax.axis_index("i")

        q = lax.rem(d, 4)
        z = lax.div(d, 4)
        p = jnp.where(q == 0, z,
            jnp.where(q == 1, 7 - z,
            jnp.where(q == 2, 8 + z, 15 - z)))

        def ring_dev(pos):
            pos = lax.rem(pos + 2 * N_DEV, N_DEV)
            c = lax.div(pos, 4)
            r = lax.rem(pos, 4)
            zz = jnp.where(lax.rem(c, 2) == 1, 3 - r, r)
            return 4 * zz + c

        right = ring_dev(p + 1)
        left = ring_dev(p - 1)

        barrier_sem = pltpu.get_barrier_semaphore()
        for nbr in (left, right):
            pl.semaphore_signal(
                barrier_sem, inc=1,
                device_id=(nbr,), device_id_type=pl.DeviceIdType.MESH,
            )
        pl.semaphore_wait(barrier_sem, 2)

        def chunk_cols(c, dirn, g):
            col0 = dirn * HALF + g * GW
            return part_ref[pl.ds(c * M_PER, M_PER), col0:col0 + GW]

        def send_chunk_idx(dirn, s):
            if dirn == 0:
                return ring_dev(p - 1 - s)
            return ring_dev(p + 1 + s)

        def rdma(s, dirn, g):
            return pltpu.make_async_remote_copy(
                src_ref=send_ref.at[s, dirn, g],
                dst_ref=recv_ref.at[s, dirn, g],
                send_sem=send_sems.at[s, dirn, g],
                recv_sem=recv_sems.at[s, dirn, g],
                device_id=(right if dirn == 0 else left,),
                device_id_type=pl.DeviceIdType.MESH,
            )

        for dirn in (0, 1):
            c = send_chunk_idx(dirn, 0)
            p0 = jnp.dot(
                x_ref[pl.ds(c * M_PER, M_PER), :], w_ref[...],
                preferred_element_type=jnp.float32,
            ).astype(jnp.bfloat16)
            for g in range(G):
                col0 = dirn * HALF + g * GW
                send_ref[0, dirn, g] = p0[:, col0:col0 + GW]
            for g in range(G):
                rdma(0, dirn, g).start()

        part_ref[...] = jnp.dot(
            x_ref[...], w_ref[...], preferred_element_type=jnp.float32
        ).astype(jnp.bfloat16)

        for s in range(1, N_HOP):
            for g in range(G):
                for dirn in (0, 1):
                    rdma(s - 1, dirn, g).wait_recv()
                    send_ref[s, dirn, g] = (
                        recv_ref[s - 1, dirn, g]
                        + chunk_cols(send_chunk_idx(dirn, s), dirn, g)
                    )
                    rdma(s, dirn, g).start()

        for dirn in (0, 1):
            for g in range(G):
                rdma(N_HOP - 1, dirn, g).wait_recv()
                col0 = dirn * HALF + g * GW
                y = (recv_ref[N_HOP - 1, dirn, g].astype(jnp.float32)
                     + chunk_cols(d, dirn, g).astype(jnp.float32))
                out_ref[:, col0:col0 + GW] = _gelu(y)

        for s in range(N_HOP):
            for dirn in (0, 1):
                for g in range(G):
                    rdma(s, dirn, g).wait_send()

    return pl.pallas_call(
        body,
        out_shape=jax.ShapeDtypeStruct((M_PER, N), jnp.float32),
        in_specs=[
            pl.BlockSpec(memory_space=pltpu.VMEM),
            pl.BlockSpec(memory_space=pltpu.VMEM),
        ],
        out_specs=pl.BlockSpec(memory_space=pltpu.VMEM),
        scratch_shapes=[
            pltpu.VMEM((M, N), jnp.bfloat16),
            pltpu.VMEM((N_HOP, 2, G, M_PER, GW), jnp.bfloat16),
            pltpu.VMEM((N_HOP, 2, G, M_PER, GW), jnp.bfloat16),
            pltpu.SemaphoreType.DMA((N_HOP, 2, G)),
            pltpu.SemaphoreType.DMA((N_HOP, 2, G)),
        ],
        compiler_params=pltpu.CompilerParams(collective_id=0),
    )(x, w_mat)
